# baseline (speedup 1.0000x reference)
import jax
import jax.numpy as jnp
from jax import lax
from jax.experimental import pallas as pl
from jax.experimental.pallas import tpu as pltpu
from jax.experimental.pallas import tpu_sc as plsc

B = 16384
D = 32
NROWS = 1000000

_info = plsc.get_sparse_core_info()
NC, NS, L = _info.num_cores, _info.num_subcores, _info.num_lanes
NW = NC * NS
BPW = B // NW
NG = BPW // L
GPW = BPW * D
CHUNK = 128
NCH = GPW // CHUNK
NBCH = BPW // CHUNK


def _mf_body(users_h, items_h, uflat_h, iflat_h, ub_h, ib_h, gb_h,
             scores_h, reg_h,
             idx_u, idx_i, offs_u, offs_i, urows, irows, ub, ib,
             gb_v, scores_v, reg_v, sem):
    wid = lax.axis_index("s") * NC + lax.axis_index("c")
    base = wid * BPW

    pltpu.sync_copy(users_h.at[pl.ds(base, BPW)], idx_u)
    pltpu.sync_copy(items_h.at[pl.ds(base, BPW)], idx_i)
    pltpu.sync_copy(gb_h, gb_v)

    def mkoff(g, _):
        lanes = pl.ds(g * L, L)
        us = idx_u[lanes]
        its = idx_i[lanes]
        for d in range(D):
            offs_u[pl.ds(d * BPW + g * L, L)] = us + d * NROWS
            offs_i[pl.ds(d * BPW + g * L, L)] = its + d * NROWS
        return 0

    lax.fori_loop(0, NG, mkoff, 0)

    def fire(c, _):
        s = pl.ds(c * CHUNK, CHUNK)
        pltpu.async_copy(uflat_h.at[offs_u.at[s]], urows.at[s], sem)
        pltpu.async_copy(iflat_h.at[offs_i.at[s]], irows.at[s], sem)
        return 0

    lax.fori_loop(0, NCH, fire, 0)
    for j in range(NBCH):
        s = pl.ds(j * CHUNK, CHUNK)
        pltpu.async_copy(ub_h.at[idx_u.at[s]], ub.at[s], sem)
        pltpu.async_copy(ib_h.at[idx_i.at[s]], ib.at[s], sem)

    pltpu.make_async_copy(uflat_h.at[pl.ds(0, GPW)], urows, sem).wait()
    pltpu.make_async_copy(iflat_h.at[pl.ds(0, GPW)], irows, sem).wait()
    pltpu.make_async_copy(ub_h.at[pl.ds(0, BPW)], ub, sem).wait()
    pltpu.make_async_copy(ib_h.at[pl.ds(0, BPW)], ib, sem).wait()

    gb = gb_v[...]

    def group(g, racc):
        lanes = pl.ds(g * L, L)
        acc = jnp.zeros((L,), jnp.float32)
        for d in range(D):
            uv = urows[pl.ds(d * BPW + g * L, L)]
            iv = irows[pl.ds(d * BPW + g * L, L)]
            acc = acc + uv * iv
            racc = racc + uv * uv + iv * iv
        ubv = ub[lanes]
        ibv = ib[lanes]
        racc = racc + ubv * ubv + ibv * ibv
        x = acc + ubv + ibv + gb
        scores_v[lanes] = 1.0 / (1.0 + jnp.exp(-x))
        return racc

    racc = lax.fori_loop(0, NG, group, jnp.zeros((L,), jnp.float32))
    reg_v[...] = racc
    pltpu.sync_copy(scores_v, scores_h.at[pl.ds(base, BPW)])
    pltpu.sync_copy(reg_v, reg_h.at[wid])


def kernel(users, items, user_emb, item_emb, user_bias, item_bias, global_bias):
    users = users.astype(jnp.int32)
    items = items.astype(jnp.int32)
    uflat = user_emb.T.reshape(D * NROWS)
    iflat = item_emb.T.reshape(D * NROWS)
    ub1 = user_bias.reshape(NROWS)
    ib1 = item_bias.reshape(NROWS)
    gb_vec = jnp.broadcast_to(global_bias.astype(jnp.float32), (L,))
    mesh = plsc.VectorSubcoreMesh(core_axis_name="c", subcore_axis_name="s")
    run = pl.kernel(
        _mf_body,
        mesh=mesh,
        compiler_params=pltpu.CompilerParams(
            use_tc_tiling_on_sc=False, needs_layout_passes=False),
        out_type=[
            jax.ShapeDtypeStruct((B,), jnp.float32),
            jax.ShapeDtypeStruct((NW, L), jnp.float32),
        ],
        scratch_types=[
            pltpu.VMEM((BPW,), jnp.int32),
            pltpu.VMEM((BPW,), jnp.int32),
            pltpu.VMEM((GPW,), jnp.int32),
            pltpu.VMEM((GPW,), jnp.int32),
            pltpu.VMEM((GPW,), jnp.float32),
            pltpu.VMEM((GPW,), jnp.float32),
            pltpu.VMEM((BPW,), jnp.float32),
            pltpu.VMEM((BPW,), jnp.float32),
            pltpu.VMEM((L,), jnp.float32),
            pltpu.VMEM((BPW,), jnp.float32),
            pltpu.VMEM((L,), jnp.float32),
            pltpu.SemaphoreType.DMA,
        ],
    )
    scores, reg_parts = run(users, items, uflat, iflat, ub1, ib1, gb_vec)
    regularizer = jnp.sum(reg_parts) / jnp.float32(B)
    return scores, regularizer


# speedup vs baseline: 5.7027x; 5.7027x over previous
import jax
import jax.numpy as jnp
from jax import lax
from jax.experimental import pallas as pl
from jax.experimental.pallas import tpu as pltpu
from jax.experimental.pallas import tpu_sc as plsc

B = 16384
D = 32
NROWS = 1000000
RPT = 128 // D

_info = plsc.get_sparse_core_info()
NC, NS, L = _info.num_cores, _info.num_subcores, _info.num_lanes
NW = NC * NS
BPW = B // NW
CHUNK = 128
NCHUNK = BPW // CHUNK
NGC = CHUNK // L


def _mf_body(users_h, items_h, uemb_h, iemb_h, ub_h, ib_h, gb_h,
             scores_h, reg_h,
             idx_u, idx_i, idxr_u, idxr_i, uchunk, ichunk, ub, ib,
             gb_v, scores_v, reg_v, sem):
    wid = lax.axis_index("s") * NC + lax.axis_index("c")
    base = wid * BPW

    cps = []
    for j in range(NCHUNK):
        src_u = users_h.at[pl.ds(base + j * CHUNK, CHUNK)]
        src_i = items_h.at[pl.ds(base + j * CHUNK, CHUNK)]
        cps.append(pltpu.async_copy(src_u, idx_u.at[j], sem))
        cps.append(pltpu.async_copy(src_i, idx_i.at[j], sem))
    cps.append(pltpu.async_copy(gb_h, gb_v, sem))
    for c in cps:
        c.wait()

    iota = lax.iota(jnp.int32, L)
    for j in range(NCHUNK):
        for v in range(CHUNK // L):
            lanes = pl.ds(v * L, L)
            idxr_u[j, lanes] = lax.shift_right_logical(idx_u[j, lanes], 2)
            idxr_i[j, lanes] = lax.shift_right_logical(idx_i[j, lanes], 2)

    bps = []
    for j in range(NCHUNK):
        s = pl.ds(j * CHUNK, CHUNK)
        bps.append(pltpu.async_copy(ub_h.at[idx_u.at[j]], ub.at[s], sem))
        bps.append(pltpu.async_copy(ib_h.at[idx_i.at[j]], ib.at[s], sem))

    def chunk_dot(j, racc):
        pltpu.async_copy(uemb_h.at[idxr_u.at[j]], uchunk, sem)
        pltpu.async_copy(iemb_h.at[idxr_i.at[j]], ichunk, sem)
        pltpu.make_async_copy(uemb_h.at[pl.ds(0, CHUNK)], uchunk, sem).wait()
        pltpu.make_async_copy(iemb_h.at[pl.ds(0, CHUNK)], ichunk, sem).wait()
        for g in range(NGC):
            lanes = pl.ds(g * L, L)
            rows = g * L + iota
            us = idx_u[j, lanes]
            its = idx_i[j, lanes]
            ubase = lax.bitwise_and(us, RPT - 1) * D
            ibase = lax.bitwise_and(its, RPT - 1) * D
            acc = jnp.zeros((L,), jnp.float32)
            for k in range(D):
                col = lax.bitwise_and(iota + k, D - 1)
                uv = plsc.load_gather(uchunk, [rows, ubase + col])
                iv = plsc.load_gather(ichunk, [rows, ibase + col])
                acc = acc + uv * iv
                racc = racc + uv * uv + iv * iv
            scores_v[pl.ds(j * CHUNK + g * L, L)] = acc
        return racc

    racc = lax.fori_loop(0, NCHUNK, chunk_dot, jnp.zeros((L,), jnp.float32))

    for c in bps:
        c.wait()
    gb = gb_v[...]
    for t in range(BPW // L):
        lanes = pl.ds(t * L, L)
        ubv = ub[lanes]
        ibv = ib[lanes]
        racc = racc + ubv * ubv + ibv * ibv
        x = scores_v[lanes] + ubv + ibv + gb
        scores_v[lanes] = 1.0 / (1.0 + jnp.exp(-x))

    reg_v[...] = racc
    pltpu.sync_copy(scores_v, scores_h.at[pl.ds(base, BPW)])
    pltpu.sync_copy(reg_v, reg_h.at[wid])


def kernel(users, items, user_emb, item_emb, user_bias, item_bias, global_bias):
    users = users.astype(jnp.int32)
    items = items.astype(jnp.int32)
    uview = user_emb.reshape(NROWS // RPT, RPT * D)
    iview = item_emb.reshape(NROWS // RPT, RPT * D)
    ub1 = user_bias.reshape(NROWS)
    ib1 = item_bias.reshape(NROWS)
    gb_vec = jnp.broadcast_to(global_bias.astype(jnp.float32), (L,))
    mesh = plsc.VectorSubcoreMesh(core_axis_name="c", subcore_axis_name="s")
    run = pl.kernel(
        _mf_body,
        mesh=mesh,
        compiler_params=pltpu.CompilerParams(
            use_tc_tiling_on_sc=True, needs_layout_passes=False),
        out_type=[
            jax.ShapeDtypeStruct((B,), jnp.float32),
            jax.ShapeDtypeStruct((NW, L), jnp.float32),
        ],
        scratch_types=[
            pltpu.VMEM((NCHUNK, CHUNK), jnp.int32),
            pltpu.VMEM((NCHUNK, CHUNK), jnp.int32),
            pltpu.VMEM((NCHUNK, CHUNK), jnp.int32),
            pltpu.VMEM((NCHUNK, CHUNK), jnp.int32),
            pltpu.VMEM((CHUNK, RPT * D), jnp.float32),
            pltpu.VMEM((CHUNK, RPT * D), jnp.float32),
            pltpu.VMEM((BPW,), jnp.float32),
            pltpu.VMEM((BPW,), jnp.float32),
            pltpu.VMEM((L,), jnp.float32),
            pltpu.VMEM((BPW,), jnp.float32),
            pltpu.VMEM((L,), jnp.float32),
            pltpu.SemaphoreType.DMA,
        ],
    )
    scores, reg_parts = run(users, items, uview, iview, ub1, ib1, gb_vec)
    regularizer = jnp.sum(reg_parts) / jnp.float32(B)
    return scores, regularizer


# speedup vs baseline: 5.7737x; 1.0124x over previous
import jax
import jax.numpy as jnp
from jax import lax
from jax.experimental import pallas as pl
from jax.experimental.pallas import tpu as pltpu
from jax.experimental.pallas import tpu_sc as plsc

B = 16384
D = 32
NROWS = 1000000

_info = plsc.get_sparse_core_info()
NC, NS, L = _info.num_cores, _info.num_subcores, _info.num_lanes
NW = NC * NS
BPW = B // NW
CHUNK = 128
NCHUNK = BPW // CHUNK
NG = BPW // L


def _mf_body(users_h, items_h, uemb_h, iemb_h, ub_h, ib_h, gb_h,
             scores_h, reg_h,
             idx_u, idx_i, urows, irows, ub, ib, gb_v, scores_v, reg_v, sem):
    wid = lax.axis_index("s") * NC + lax.axis_index("c")
    base = wid * BPW

    cps = []
    for j in range(NCHUNK):
        src_u = users_h.at[pl.ds(base + j * CHUNK, CHUNK)]
        src_i = items_h.at[pl.ds(base + j * CHUNK, CHUNK)]
        cps.append(pltpu.async_copy(src_u, idx_u.at[j], sem))
        cps.append(pltpu.async_copy(src_i, idx_i.at[j], sem))
    cps.append(pltpu.async_copy(gb_h, gb_v, sem))
    for c in cps:
        c.wait()

    cps = []
    for j in range(NCHUNK):
        s = pl.ds(j * CHUNK, CHUNK)
        cps.append(pltpu.async_copy(uemb_h.at[idx_u.at[j]], urows.at[s], sem))
        cps.append(pltpu.async_copy(iemb_h.at[idx_i.at[j]], irows.at[s], sem))
        cps.append(pltpu.async_copy(ub_h.at[idx_u.at[j]], ub.at[s], sem))
        cps.append(pltpu.async_copy(ib_h.at[idx_i.at[j]], ib.at[s], sem))
    for c in cps:
        c.wait()

    iota = lax.iota(jnp.int32, L)
    gb = gb_v[...]

    def group(g, racc):
        rows = g * L + iota
        acc = jnp.zeros((L,), jnp.float32)
        for k in range(D):
            col = lax.bitwise_and(iota + k, D - 1)
            uv = plsc.load_gather(urows, [rows, col])
            iv = plsc.load_gather(irows, [rows, col])
            acc = acc + uv * iv
            racc = racc + uv * uv + iv * iv
        ubv = plsc.load_gather(ub, [rows])
        ibv = plsc.load_gather(ib, [rows])
        racc = racc + ubv * ubv + ibv * ibv
        x = acc + ubv + ibv + gb
        s = 1.0 / (1.0 + jnp.exp(-x))
        plsc.store_scatter(scores_v, [rows], s)
        return racc

    racc = lax.fori_loop(0, NG, group, jnp.zeros((L,), jnp.float32))
    reg_v[...] = racc
    pltpu.sync_copy(scores_v, scores_h.at[pl.ds(base, BPW)])
    pltpu.sync_copy(reg_v, reg_h.at[wid])


def kernel(users, items, user_emb, item_emb, user_bias, item_bias, global_bias):
    users = users.astype(jnp.int32)
    items = items.astype(jnp.int32)
    ub1 = user_bias.reshape(NROWS)
    ib1 = item_bias.reshape(NROWS)
    gb_vec = jnp.broadcast_to(global_bias.astype(jnp.float32), (L,))
    mesh = plsc.VectorSubcoreMesh(core_axis_name="c", subcore_axis_name="s")
    run = pl.kernel(
        _mf_body,
        mesh=mesh,
        compiler_params=pltpu.CompilerParams(
            use_tc_tiling_on_sc=False, needs_layout_passes=False),
        out_type=[
            jax.ShapeDtypeStruct((B,), jnp.float32),
            jax.ShapeDtypeStruct((NW, L), jnp.float32),
        ],
        scratch_types=[
            pltpu.VMEM((NCHUNK, CHUNK), jnp.int32),
            pltpu.VMEM((NCHUNK, CHUNK), jnp.int32),
            pltpu.VMEM((BPW, D), jnp.float32),
            pltpu.VMEM((BPW, D), jnp.float32),
            pltpu.VMEM((BPW,), jnp.float32),
            pltpu.VMEM((BPW,), jnp.float32),
            pltpu.VMEM((L,), jnp.float32),
            pltpu.VMEM((BPW,), jnp.float32),
            pltpu.VMEM((L,), jnp.float32),
            pltpu.SemaphoreType.DMA,
        ],
    )
    scores, reg_parts = run(users, items, user_emb, item_emb,
                            ub1, ib1, gb_vec)
    regularizer = jnp.sum(reg_parts) / jnp.float32(B)
    return scores, regularizer
